# baseline (speedup 1.0000x reference)
import jax
import jax.numpy as jnp
from jax import lax
from jax.experimental import pallas as pl
from jax.experimental.pallas import tpu as pltpu
from jax.experimental.pallas import tpu_sc as plsc

_B = 16384
_NC = 2
_NS = 16
_NW = _NC * _NS
_LANES = 16
_BPW = _B // _NW
_D = 1.702
_VALUE_RANGE = 8.0
_A_RANGE = 3.0


def _sigmoid(x):
    return 1.0 / (1.0 + jnp.exp(-x))


def _abc_body(item_hbm, a_hbm, b_hbm, c_hbm,
              a_out, b_out, c_out,
              iidx_v, a_v, b_v, c_v, sem):
    wid = lax.axis_index("s") * _NC + lax.axis_index("c")
    base = wid * _BPW
    sl = pl.ds(base, _BPW)
    pltpu.sync_copy(item_hbm.at[sl], iidx_v)
    cps = [
        pltpu.async_copy(a_hbm.at[iidx_v], a_v, sem),
        pltpu.async_copy(b_hbm.at[iidx_v], b_v, sem),
        pltpu.async_copy(c_hbm.at[iidx_v], c_v, sem),
    ]
    for cp in cps:
        cp.wait()
    pltpu.sync_copy(a_v, a_out.at[sl])
    pltpu.sync_copy(b_v, b_out.at[sl])
    pltpu.sync_copy(c_v, c_out.at[sl])


_sc_abc = pl.kernel(
    _abc_body,
    out_type=[jax.ShapeDtypeStruct((_B,), jnp.float32)] * 3,
    mesh=plsc.VectorSubcoreMesh(core_axis_name="c", subcore_axis_name="s"),
    scratch_types=[
        pltpu.VMEM((_BPW,), jnp.int32),
        pltpu.VMEM((_BPW,), jnp.float32),
        pltpu.VMEM((_BPW,), jnp.float32),
        pltpu.VMEM((_BPW,), jnp.float32),
        pltpu.SemaphoreType.DMA,
    ],
)


def _theta_irf_body(user_hbm, theta_hbm, ag_hbm, bg_hbm, cg_hbm,
                    irf_hbm, theta_raw_hbm,
                    uidx_v, th_v, a_v, b_v, c_v, irf_v, sem):
    wid = lax.axis_index("s") * _NC + lax.axis_index("c")
    base = wid * _BPW
    sl = pl.ds(base, _BPW)
    pltpu.sync_copy(user_hbm.at[sl], uidx_v)
    cp = pltpu.async_copy(theta_hbm.at[uidx_v], th_v, sem)
    pltpu.sync_copy(ag_hbm.at[sl], a_v)
    pltpu.sync_copy(bg_hbm.at[sl], b_v)
    pltpu.sync_copy(cg_hbm.at[sl], c_v)
    cp.wait()

    def step(i, _):
        s = pl.ds(i * _LANES, _LANES)
        th = th_v[s]
        a = a_v[s]
        b = b_v[s]
        c = c_v[s]
        c_s = _sigmoid(c)
        theta_t = _VALUE_RANGE * (_sigmoid(th) - 0.5)
        b_t = _VALUE_RANGE * (_sigmoid(b) - 0.5)
        a_t = _A_RANGE * _sigmoid(a)
        irf = c_s + (1.0 - c_s) / (1.0 + jnp.exp(-_D * a_t * (theta_t - b_t)))
        irf_v[s] = irf
        return 0

    lax.fori_loop(0, _BPW // _LANES, step, 0)
    pltpu.sync_copy(irf_v, irf_hbm.at[sl])
    pltpu.sync_copy(th_v, theta_raw_hbm.at[sl])


_sc_theta_irf = pl.kernel(
    _theta_irf_body,
    out_type=[
        jax.ShapeDtypeStruct((_B,), jnp.float32),
        jax.ShapeDtypeStruct((_B,), jnp.float32),
    ],
    mesh=plsc.VectorSubcoreMesh(core_axis_name="c", subcore_axis_name="s"),
    scratch_types=[
        pltpu.VMEM((_BPW,), jnp.int32),
        pltpu.VMEM((_BPW,), jnp.float32),
        pltpu.VMEM((_BPW,), jnp.float32),
        pltpu.VMEM((_BPW,), jnp.float32),
        pltpu.VMEM((_BPW,), jnp.float32),
        pltpu.VMEM((_BPW,), jnp.float32),
        pltpu.SemaphoreType.DMA,
    ],
)


_MLP_BLK = 2048


def _mlp_body(x_ref, w1_ref, b1_ref, w2_ref, b2_ref, w3_ref, b3_ref,
              wd_ref, bd_ref, d_ref):
    x = x_ref[...].reshape(_MLP_BLK, 1)
    h1 = jnp.maximum(x * w1_ref[...] + b1_ref[...], 0.0)
    h2 = jnp.maximum(
        jnp.dot(h1, w2_ref[...], preferred_element_type=jnp.float32)
        + b2_ref[...], 0.0)
    h3 = jnp.dot(h2, w3_ref[...], preferred_element_type=jnp.float32)
    d_ref[...] = (h3 + b3_ref[0, 0]) * wd_ref[0, 0] + bd_ref[0, 0]


_mlp = pl.pallas_call(
    _mlp_body,
    grid=(_B // _MLP_BLK,),
    in_specs=[
        pl.BlockSpec((_MLP_BLK,), lambda i: (i,)),
        pl.BlockSpec((1, 64), lambda i: (0, 0)),
        pl.BlockSpec((1, 64), lambda i: (0, 0)),
        pl.BlockSpec((64, 32), lambda i: (0, 0)),
        pl.BlockSpec((1, 32), lambda i: (0, 0)),
        pl.BlockSpec((32, 1), lambda i: (0, 0)),
        pl.BlockSpec((1, 1), lambda i: (0, 0)),
        pl.BlockSpec((1, 1), lambda i: (0, 0)),
        pl.BlockSpec((1, 1), lambda i: (0, 0)),
    ],
    out_specs=pl.BlockSpec((_MLP_BLK, 1), lambda i: (i, 0)),
    out_shape=jax.ShapeDtypeStruct((_B, 1), jnp.float32),
)


def kernel(user, item, theta_table, a_table, b_table, c_table,
           W1, b1, W2, b2, W3, b3, Wd, bd):
    a_g, b_g, c_g = _sc_abc(
        item, a_table.reshape(-1), b_table.reshape(-1), c_table.reshape(-1))
    irf_out, theta_raw = _sc_theta_irf(
        user, theta_table.reshape(-1), a_g, b_g, c_g)
    d_output = _mlp(
        theta_raw,
        W1.reshape(1, 64), b1.reshape(1, 64),
        W2.T, b2.reshape(1, 32),
        W3.T, b3.reshape(1, 1),
        Wd, bd.reshape(1, 1))
    return (irf_out, d_output)


# speedup vs baseline: 1.0197x; 1.0197x over previous
import jax
import jax.numpy as jnp
from jax import lax
from jax.experimental import pallas as pl
from jax.experimental.pallas import tpu as pltpu
from jax.experimental.pallas import tpu_sc as plsc

_B = 16384
_NC = 2
_NS = 16
_NW = _NC * _NS
_LANES = 16
_BPW = _B // _NW
_D = 1.702
_VALUE_RANGE = 8.0
_A_RANGE = 3.0


def _sigmoid(x):
    return 1.0 / (1.0 + jnp.exp(-x))


def _abc_body(item_hbm, a_hbm, b_hbm, c_hbm,
              a_out, b_out, c_out,
              iidx_v, a_v, b_v, c_v, sem):
    wid = lax.axis_index("s") * _NC + lax.axis_index("c")
    base = wid * _BPW
    sl = pl.ds(base, _BPW)
    pltpu.sync_copy(item_hbm.at[sl], iidx_v)
    cps = [
        pltpu.async_copy(a_hbm.at[iidx_v], a_v, sem),
        pltpu.async_copy(b_hbm.at[iidx_v], b_v, sem),
        pltpu.async_copy(c_hbm.at[iidx_v], c_v, sem),
    ]
    for cp in cps:
        cp.wait()
    pltpu.sync_copy(a_v, a_out.at[sl])
    pltpu.sync_copy(b_v, b_out.at[sl])
    pltpu.sync_copy(c_v, c_out.at[sl])


_sc_abc = pl.kernel(
    _abc_body,
    out_type=[jax.ShapeDtypeStruct((_B,), jnp.float32)] * 3,
    mesh=plsc.VectorSubcoreMesh(core_axis_name="c", subcore_axis_name="s"),
    scratch_types=[
        pltpu.VMEM((_BPW,), jnp.int32),
        pltpu.VMEM((_BPW,), jnp.float32),
        pltpu.VMEM((_BPW,), jnp.float32),
        pltpu.VMEM((_BPW,), jnp.float32),
        pltpu.SemaphoreType.DMA,
    ],
)


def _theta_irf_body(user_hbm, theta_hbm, ag_hbm, bg_hbm, cg_hbm,
                    irf_hbm, theta_raw_hbm,
                    uidx_v, th_v, a_v, b_v, c_v, irf_v, sem):
    wid = lax.axis_index("s") * _NC + lax.axis_index("c")
    base = wid * _BPW
    sl = pl.ds(base, _BPW)
    pltpu.sync_copy(user_hbm.at[sl], uidx_v)
    cp = pltpu.async_copy(theta_hbm.at[uidx_v], th_v, sem)
    pltpu.sync_copy(ag_hbm.at[sl], a_v)
    pltpu.sync_copy(bg_hbm.at[sl], b_v)
    pltpu.sync_copy(cg_hbm.at[sl], c_v)
    cp.wait()

    def step(i, _):
        s = pl.ds(i * _LANES, _LANES)
        th = th_v[s]
        a = a_v[s]
        b = b_v[s]
        c = c_v[s]
        c_s = _sigmoid(c)
        theta_t = _VALUE_RANGE * (_sigmoid(th) - 0.5)
        b_t = _VALUE_RANGE * (_sigmoid(b) - 0.5)
        a_t = _A_RANGE * _sigmoid(a)
        irf = c_s + (1.0 - c_s) / (1.0 + jnp.exp(-_D * a_t * (theta_t - b_t)))
        irf_v[s] = irf
        return 0

    lax.fori_loop(0, _BPW // _LANES, step, 0)
    pltpu.sync_copy(irf_v, irf_hbm.at[sl])
    pltpu.sync_copy(th_v, theta_raw_hbm.at[sl])


_sc_theta_irf = pl.kernel(
    _theta_irf_body,
    out_type=[
        jax.ShapeDtypeStruct((_B,), jnp.float32),
        jax.ShapeDtypeStruct((_B,), jnp.float32),
    ],
    mesh=plsc.VectorSubcoreMesh(core_axis_name="c", subcore_axis_name="s"),
    scratch_types=[
        pltpu.VMEM((_BPW,), jnp.int32),
        pltpu.VMEM((_BPW,), jnp.float32),
        pltpu.VMEM((_BPW,), jnp.float32),
        pltpu.VMEM((_BPW,), jnp.float32),
        pltpu.VMEM((_BPW,), jnp.float32),
        pltpu.VMEM((_BPW,), jnp.float32),
        pltpu.SemaphoreType.DMA,
    ],
)


_MLP_BLK = 2048


def _mlp_body(x_ref, w1_ref, b1_ref, w2_ref, b2_ref, w3_ref, b3_ref,
              wd_ref, bd_ref, d_ref):
    x = x_ref[...].reshape(_MLP_BLK, 1)
    h1 = jnp.maximum(x * w1_ref[...] + b1_ref[...], 0.0)
    h2 = jnp.maximum(
        jnp.dot(h1, w2_ref[...], preferred_element_type=jnp.float32)
        + b2_ref[...], 0.0)
    h3 = jnp.dot(h2, w3_ref[...], preferred_element_type=jnp.float32)
    d = (h3 + b3_ref[0, 0]) * wd_ref[0, 0] + bd_ref[0, 0]
    d_ref[...] = d.reshape(1, _MLP_BLK)


_mlp = pl.pallas_call(
    _mlp_body,
    grid=(_B // _MLP_BLK,),
    in_specs=[
        pl.BlockSpec((_MLP_BLK,), lambda i: (i,)),
        pl.BlockSpec((1, 64), lambda i: (0, 0)),
        pl.BlockSpec((1, 64), lambda i: (0, 0)),
        pl.BlockSpec((64, 32), lambda i: (0, 0)),
        pl.BlockSpec((1, 32), lambda i: (0, 0)),
        pl.BlockSpec((32, 1), lambda i: (0, 0)),
        pl.BlockSpec((1, 1), lambda i: (0, 0)),
        pl.BlockSpec((1, 1), lambda i: (0, 0)),
        pl.BlockSpec((1, 1), lambda i: (0, 0)),
    ],
    out_specs=pl.BlockSpec((1, _MLP_BLK), lambda i: (0, i)),
    out_shape=jax.ShapeDtypeStruct((1, _B), jnp.float32),
)


def kernel(user, item, theta_table, a_table, b_table, c_table,
           W1, b1, W2, b2, W3, b3, Wd, bd):
    a_g, b_g, c_g = _sc_abc(
        item, a_table.reshape(-1), b_table.reshape(-1), c_table.reshape(-1))
    irf_out, theta_raw = _sc_theta_irf(
        user, theta_table.reshape(-1), a_g, b_g, c_g)
    d_r = _mlp(
        theta_raw,
        W1.reshape(1, 64), b1.reshape(1, 64),
        W2.T, b2.reshape(1, 32),
        W3.T, b3.reshape(1, 1),
        Wd, bd.reshape(1, 1))
    return (irf_out, d_r.reshape(_B, 1))


# speedup vs baseline: 1.1369x; 1.1150x over previous
import jax
import jax.numpy as jnp
from jax import lax
from jax.experimental import pallas as pl
from jax.experimental.pallas import tpu as pltpu
from jax.experimental.pallas import tpu_sc as plsc

_B = 16384
_NC = 2
_NS = 16
_NW = _NC * _NS
_LANES = 16
_BPW = _B // _NW
_D = 1.702
_VALUE_RANGE = 8.0
_A_RANGE = 3.0


def _sigmoid(x):
    return 1.0 / (1.0 + jnp.exp(-x))


def _abc_body(item_hbm, a_hbm, b_hbm, c_hbm,
              a_out, b_out, c_out,
              iidx_v, a_v, b_v, c_v, sem):
    wid = lax.axis_index("s") * _NC + lax.axis_index("c")
    base = wid * _BPW
    sl = pl.ds(base, _BPW)
    pltpu.sync_copy(item_hbm.at[sl], iidx_v)
    cps = [
        pltpu.async_copy(a_hbm.at[iidx_v], a_v, sem),
        pltpu.async_copy(b_hbm.at[iidx_v], b_v, sem),
        pltpu.async_copy(c_hbm.at[iidx_v], c_v, sem),
    ]
    for cp in cps:
        cp.wait()
    pltpu.sync_copy(a_v, a_out.at[sl])
    pltpu.sync_copy(b_v, b_out.at[sl])
    pltpu.sync_copy(c_v, c_out.at[sl])


_sc_abc = pl.kernel(
    _abc_body,
    out_type=[jax.ShapeDtypeStruct((_B,), jnp.float32)] * 3,
    mesh=plsc.VectorSubcoreMesh(core_axis_name="c", subcore_axis_name="s"),
    scratch_types=[
        pltpu.VMEM((_BPW,), jnp.int32),
        pltpu.VMEM((_BPW,), jnp.float32),
        pltpu.VMEM((_BPW,), jnp.float32),
        pltpu.VMEM((_BPW,), jnp.float32),
        pltpu.SemaphoreType.DMA,
    ],
)


def _theta_irf_body(user_hbm, theta_hbm, ag_hbm, bg_hbm, cg_hbm,
                    irf_hbm, theta_raw_hbm,
                    uidx_v, th_v, a_v, b_v, c_v, irf_v, sem):
    wid = lax.axis_index("s") * _NC + lax.axis_index("c")
    base = wid * _BPW
    sl = pl.ds(base, _BPW)
    pltpu.sync_copy(user_hbm.at[sl], uidx_v)
    cp = pltpu.async_copy(theta_hbm.at[uidx_v], th_v, sem)
    pltpu.sync_copy(ag_hbm.at[sl], a_v)
    pltpu.sync_copy(bg_hbm.at[sl], b_v)
    pltpu.sync_copy(cg_hbm.at[sl], c_v)
    cp.wait()

    def step(i, _):
        s = pl.ds(i * _LANES, _LANES)
        th = th_v[s]
        a = a_v[s]
        b = b_v[s]
        c = c_v[s]
        c_s = _sigmoid(c)
        theta_t = _VALUE_RANGE * (_sigmoid(th) - 0.5)
        b_t = _VALUE_RANGE * (_sigmoid(b) - 0.5)
        a_t = _A_RANGE * _sigmoid(a)
        irf = c_s + (1.0 - c_s) / (1.0 + jnp.exp(-_D * a_t * (theta_t - b_t)))
        irf_v[s] = irf
        return 0

    lax.fori_loop(0, _BPW // _LANES, step, 0)
    pltpu.sync_copy(irf_v, irf_hbm.at[sl])
    pltpu.sync_copy(th_v, theta_raw_hbm.at[sl])


_sc_theta_irf = pl.kernel(
    _theta_irf_body,
    out_type=[
        jax.ShapeDtypeStruct((_B,), jnp.float32),
        jax.ShapeDtypeStruct((_B,), jnp.float32),
    ],
    mesh=plsc.VectorSubcoreMesh(core_axis_name="c", subcore_axis_name="s"),
    scratch_types=[
        pltpu.VMEM((_BPW,), jnp.int32),
        pltpu.VMEM((_BPW,), jnp.float32),
        pltpu.VMEM((_BPW,), jnp.float32),
        pltpu.VMEM((_BPW,), jnp.float32),
        pltpu.VMEM((_BPW,), jnp.float32),
        pltpu.VMEM((_BPW,), jnp.float32),
        pltpu.SemaphoreType.DMA,
    ],
)


def _mlp_body(x_ref, w1_ref, b1_ref, w2_ref, b2_ref, w3_ref, b3_ref,
              wd_ref, bd_ref, d_ref):
    x = x_ref[...].reshape(_B, 1)
    h1 = jnp.maximum(x * w1_ref[...] + b1_ref[...], 0.0)
    h2 = jnp.maximum(
        jnp.dot(h1, w2_ref[...], preferred_element_type=jnp.float32)
        + b2_ref[...], 0.0)
    h3 = lax.dot_general(
        w3_ref[...], h2, (((1,), (1,)), ((), ())),
        preferred_element_type=jnp.float32).reshape(_B)
    d_ref[...] = (h3 + b3_ref[0, 0]) * wd_ref[0, 0] + bd_ref[0, 0]


_mlp = pl.pallas_call(
    _mlp_body,
    out_shape=jax.ShapeDtypeStruct((_B,), jnp.float32),
)


def kernel(user, item, theta_table, a_table, b_table, c_table,
           W1, b1, W2, b2, W3, b3, Wd, bd):
    a_g, b_g, c_g = _sc_abc(
        item, a_table.reshape(-1), b_table.reshape(-1), c_table.reshape(-1))
    irf_out, theta_raw = _sc_theta_irf(
        user, theta_table.reshape(-1), a_g, b_g, c_g)
    d_r = _mlp(
        theta_raw,
        W1.reshape(1, 64), b1.reshape(1, 64),
        W2.T, b2.reshape(1, 32),
        W3, b3.reshape(1, 1),
        Wd, bd.reshape(1, 1))
    return (irf_out, d_r.reshape(_B, 1))


# speedup vs baseline: 1.1396x; 1.0024x over previous
import jax
import jax.numpy as jnp
from jax import lax
from jax.experimental import pallas as pl
from jax.experimental.pallas import tpu as pltpu
from jax.experimental.pallas import tpu_sc as plsc

_B = 16384
_NC = 2
_NS = 16
_NW = _NC * _NS
_LANES = 16
_BPW = _B // _NW
_D = 1.702
_VALUE_RANGE = 8.0
_A_RANGE = 3.0


def _sigmoid(x):
    return 1.0 / (1.0 + jnp.exp(-x))


def _abc_body(item_hbm, a_hbm, b_hbm, c_hbm,
              a_out, b_out, c_out,
              iidx_v, a_v, b_v, c_v, sem):
    wid = lax.axis_index("s") * _NC + lax.axis_index("c")
    base = wid * _BPW
    sl = pl.ds(base, _BPW)
    pltpu.sync_copy(item_hbm.at[sl], iidx_v)
    cps = [
        pltpu.async_copy(a_hbm.at[iidx_v], a_v, sem),
        pltpu.async_copy(b_hbm.at[iidx_v], b_v, sem),
        pltpu.async_copy(c_hbm.at[iidx_v], c_v, sem),
    ]
    for cp in cps:
        cp.wait()
    pltpu.sync_copy(a_v, a_out.at[sl])
    pltpu.sync_copy(b_v, b_out.at[sl])
    pltpu.sync_copy(c_v, c_out.at[sl])


_sc_abc = pl.kernel(
    _abc_body,
    out_type=[jax.ShapeDtypeStruct((_B,), jnp.float32)] * 3,
    mesh=plsc.VectorSubcoreMesh(core_axis_name="c", subcore_axis_name="s"),
    scratch_types=[
        pltpu.VMEM((_BPW,), jnp.int32),
        pltpu.VMEM((_BPW,), jnp.float32),
        pltpu.VMEM((_BPW,), jnp.float32),
        pltpu.VMEM((_BPW,), jnp.float32),
        pltpu.SemaphoreType.DMA,
    ],
)


def _theta_irf_body(user_hbm, theta_hbm, ag_hbm, bg_hbm, cg_hbm,
                    irf_hbm, theta_raw_hbm,
                    uidx_v, th_v, a_v, b_v, c_v, irf_v, sem):
    wid = lax.axis_index("s") * _NC + lax.axis_index("c")
    base = wid * _BPW
    sl = pl.ds(base, _BPW)
    pltpu.sync_copy(user_hbm.at[sl], uidx_v)
    cp_th = pltpu.async_copy(theta_hbm.at[uidx_v], th_v, sem)
    cp_a = pltpu.async_copy(ag_hbm.at[sl], a_v, sem)
    cp_b = pltpu.async_copy(bg_hbm.at[sl], b_v, sem)
    cp_c = pltpu.async_copy(cg_hbm.at[sl], c_v, sem)
    cp_th.wait()
    cp_out = pltpu.async_copy(th_v, theta_raw_hbm.at[sl], sem)
    cp_a.wait()
    cp_b.wait()
    cp_c.wait()

    def step(i, _):
        s = pl.ds(i * _LANES, _LANES)
        th = th_v[s]
        a = a_v[s]
        b = b_v[s]
        c = c_v[s]
        c_s = _sigmoid(c)
        theta_t = _VALUE_RANGE * (_sigmoid(th) - 0.5)
        b_t = _VALUE_RANGE * (_sigmoid(b) - 0.5)
        a_t = _A_RANGE * _sigmoid(a)
        irf = c_s + (1.0 - c_s) / (1.0 + jnp.exp(-_D * a_t * (theta_t - b_t)))
        irf_v[s] = irf
        return 0

    lax.fori_loop(0, _BPW // _LANES, step, 0)
    pltpu.sync_copy(irf_v, irf_hbm.at[sl])
    cp_out.wait()


_sc_theta_irf = pl.kernel(
    _theta_irf_body,
    out_type=[
        jax.ShapeDtypeStruct((_B,), jnp.float32),
        jax.ShapeDtypeStruct((_B,), jnp.float32),
    ],
    mesh=plsc.VectorSubcoreMesh(core_axis_name="c", subcore_axis_name="s"),
    scratch_types=[
        pltpu.VMEM((_BPW,), jnp.int32),
        pltpu.VMEM((_BPW,), jnp.float32),
        pltpu.VMEM((_BPW,), jnp.float32),
        pltpu.VMEM((_BPW,), jnp.float32),
        pltpu.VMEM((_BPW,), jnp.float32),
        pltpu.VMEM((_BPW,), jnp.float32),
        pltpu.SemaphoreType.DMA,
    ],
)


def _mlp_body(x_ref, w1_ref, b1_ref, w2_ref, b2_ref, w3_ref, b3_ref,
              wd_ref, bd_ref, d_ref):
    x = x_ref[...].reshape(_B, 1)
    h1 = jnp.maximum(x * w1_ref[...] + b1_ref[...], 0.0)
    h2 = jnp.maximum(
        jnp.dot(h1, w2_ref[...], preferred_element_type=jnp.float32)
        + b2_ref[...], 0.0)
    h3 = lax.dot_general(
        w3_ref[...], h2, (((1,), (1,)), ((), ())),
        preferred_element_type=jnp.float32).reshape(_B)
    d_ref[...] = (h3 + b3_ref[0, 0]) * wd_ref[0, 0] + bd_ref[0, 0]


_mlp = pl.pallas_call(
    _mlp_body,
    out_shape=jax.ShapeDtypeStruct((_B,), jnp.float32),
)


def kernel(user, item, theta_table, a_table, b_table, c_table,
           W1, b1, W2, b2, W3, b3, Wd, bd):
    a_g, b_g, c_g = _sc_abc(
        item, a_table.reshape(-1), b_table.reshape(-1), c_table.reshape(-1))
    irf_out, theta_raw = _sc_theta_irf(
        user, theta_table.reshape(-1), a_g, b_g, c_g)
    d_r = _mlp(
        theta_raw,
        W1.reshape(1, 64), b1.reshape(1, 64),
        W2.T, b2.reshape(1, 32),
        W3, b3.reshape(1, 1),
        Wd, bd.reshape(1, 1))
    return (irf_out, d_r.reshape(_B, 1))


# speedup vs baseline: 1.1464x; 1.0059x over previous
import jax
import jax.numpy as jnp
from jax import lax
from jax.experimental import pallas as pl
from jax.experimental.pallas import tpu as pltpu
from jax.experimental.pallas import tpu_sc as plsc

_B = 16384
_NC = 2
_NS = 16
_NW = _NC * _NS
_BPW = _B // _NW
_D = 1.702
_VALUE_RANGE = 8.0
_A_RANGE = 3.0


def _abc_body(item_hbm, a_hbm, b_hbm, c_hbm,
              a_out, b_out, c_out,
              iidx_v, a_v, b_v, c_v, sem):
    wid = lax.axis_index("s") * _NC + lax.axis_index("c")
    base = wid * _BPW
    sl = pl.ds(base, _BPW)
    pltpu.sync_copy(item_hbm.at[sl], iidx_v)
    cps = [
        pltpu.async_copy(a_hbm.at[iidx_v], a_v, sem),
        pltpu.async_copy(b_hbm.at[iidx_v], b_v, sem),
        pltpu.async_copy(c_hbm.at[iidx_v], c_v, sem),
    ]
    for cp in cps:
        cp.wait()
    outs = [
        pltpu.async_copy(a_v, a_out.at[sl], sem),
        pltpu.async_copy(b_v, b_out.at[sl], sem),
        pltpu.async_copy(c_v, c_out.at[sl], sem),
    ]
    for cp in outs:
        cp.wait()


_sc_abc = pl.kernel(
    _abc_body,
    out_type=[jax.ShapeDtypeStruct((_B,), jnp.float32)] * 3,
    mesh=plsc.VectorSubcoreMesh(core_axis_name="c", subcore_axis_name="s"),
    scratch_types=[
        pltpu.VMEM((_BPW,), jnp.int32),
        pltpu.VMEM((_BPW,), jnp.float32),
        pltpu.VMEM((_BPW,), jnp.float32),
        pltpu.VMEM((_BPW,), jnp.float32),
        pltpu.SemaphoreType.DMA,
    ],
)


def _theta_body(user_hbm, theta_hbm, theta_out, uidx_v, th_v, sem):
    wid = lax.axis_index("s") * _NC + lax.axis_index("c")
    base = wid * _BPW
    sl = pl.ds(base, _BPW)
    pltpu.sync_copy(user_hbm.at[sl], uidx_v)
    pltpu.async_copy(theta_hbm.at[uidx_v], th_v, sem).wait()
    pltpu.sync_copy(th_v, theta_out.at[sl])


_sc_theta = pl.kernel(
    _theta_body,
    out_type=jax.ShapeDtypeStruct((_B,), jnp.float32),
    mesh=plsc.VectorSubcoreMesh(core_axis_name="c", subcore_axis_name="s"),
    scratch_types=[
        pltpu.VMEM((_BPW,), jnp.int32),
        pltpu.VMEM((_BPW,), jnp.float32),
        pltpu.SemaphoreType.DMA,
    ],
)


def _sigmoid(x):
    return 1.0 / (1.0 + jnp.exp(-x))


def _tc_body(x_ref, a_ref, b_ref, c_ref, w1_ref, b1_ref, w2_ref, b2_ref,
             w3_ref, b3_ref, wd_ref, bd_ref, irf_ref, d_ref):
    th = x_ref[...]
    c_s = _sigmoid(c_ref[...])
    theta_t = _VALUE_RANGE * (_sigmoid(th) - 0.5)
    b_t = _VALUE_RANGE * (_sigmoid(b_ref[...]) - 0.5)
    a_t = _A_RANGE * _sigmoid(a_ref[...])
    irf_ref[...] = c_s + (1.0 - c_s) / (
        1.0 + jnp.exp(-_D * a_t * (theta_t - b_t)))
    x = th.reshape(_B, 1)
    h1 = jnp.maximum(x * w1_ref[...] + b1_ref[...], 0.0)
    h2 = jnp.maximum(
        jnp.dot(h1, w2_ref[...], preferred_element_type=jnp.float32)
        + b2_ref[...], 0.0)
    h3 = lax.dot_general(
        w3_ref[...], h2, (((1,), (1,)), ((), ())),
        preferred_element_type=jnp.float32).reshape(_B)
    d_ref[...] = (h3 + b3_ref[0, 0]) * wd_ref[0, 0] + bd_ref[0, 0]


_tc_math = pl.pallas_call(
    _tc_body,
    out_shape=[
        jax.ShapeDtypeStruct((_B,), jnp.float32),
        jax.ShapeDtypeStruct((_B,), jnp.float32),
    ],
)


def kernel(user, item, theta_table, a_table, b_table, c_table,
           W1, b1, W2, b2, W3, b3, Wd, bd):
    a_g, b_g, c_g = _sc_abc(
        item, a_table.reshape(-1), b_table.reshape(-1), c_table.reshape(-1))
    theta_raw = _sc_theta(user, theta_table.reshape(-1))
    irf_out, d_out = _tc_math(
        theta_raw, a_g, b_g, c_g,
        W1.reshape(1, 64), b1.reshape(1, 64),
        W2.T, b2.reshape(1, 32),
        W3, b3.reshape(1, 1),
        Wd, bd.reshape(1, 1))
    return (irf_out, d_out.reshape(_B, 1))


# speedup vs baseline: 1.1494x; 1.0027x over previous
import jax
import jax.numpy as jnp
from jax import lax
from jax.experimental import pallas as pl
from jax.experimental.pallas import tpu as pltpu
from jax.experimental.pallas import tpu_sc as plsc

_B = 16384
_NC = 2
_NS = 16
_NW = _NC * _NS
_BPW = _B // _NW
_D = 1.702
_VALUE_RANGE = 8.0
_A_RANGE = 3.0


def _abc_body(item_hbm, a_hbm, b_hbm, c_hbm,
              a_out, b_out, c_out,
              iidx_v, a_v, b_v, c_v, sem):
    wid = lax.axis_index("s") * _NC + lax.axis_index("c")
    base = wid * _BPW
    sl = pl.ds(base, _BPW)
    pltpu.sync_copy(item_hbm.at[sl], iidx_v)
    cps = [
        pltpu.async_copy(a_hbm.at[iidx_v], a_v, sem),
        pltpu.async_copy(b_hbm.at[iidx_v], b_v, sem),
        pltpu.async_copy(c_hbm.at[iidx_v], c_v, sem),
    ]
    for cp in cps:
        cp.wait()
    outs = [
        pltpu.async_copy(a_v, a_out.at[sl], sem),
        pltpu.async_copy(b_v, b_out.at[sl], sem),
        pltpu.async_copy(c_v, c_out.at[sl], sem),
    ]
    for cp in outs:
        cp.wait()


_sc_abc = pl.kernel(
    _abc_body,
    out_type=[jax.ShapeDtypeStruct((_B,), jnp.float32)] * 3,
    mesh=plsc.VectorSubcoreMesh(core_axis_name="c", subcore_axis_name="s"),
    scratch_types=[
        pltpu.VMEM((_BPW,), jnp.int32),
        pltpu.VMEM((_BPW,), jnp.float32),
        pltpu.VMEM((_BPW,), jnp.float32),
        pltpu.VMEM((_BPW,), jnp.float32),
        pltpu.SemaphoreType.DMA,
    ],
)


def _theta_body(user_hbm, theta_hbm, theta_out, uidx_v, th_v, sem):
    wid = lax.axis_index("s") * _NC + lax.axis_index("c")
    base = wid * _BPW
    sl = pl.ds(base, _BPW)
    pltpu.sync_copy(user_hbm.at[sl], uidx_v)
    pltpu.async_copy(theta_hbm.at[uidx_v], th_v, sem).wait()
    pltpu.sync_copy(th_v, theta_out.at[sl])


_sc_theta = pl.kernel(
    _theta_body,
    out_type=jax.ShapeDtypeStruct((_B,), jnp.float32),
    mesh=plsc.VectorSubcoreMesh(core_axis_name="c", subcore_axis_name="s"),
    scratch_types=[
        pltpu.VMEM((_BPW,), jnp.int32),
        pltpu.VMEM((_BPW,), jnp.float32),
        pltpu.SemaphoreType.DMA,
    ],
)


def _sigmoid(x):
    return 1.0 / (1.0 + jnp.exp(-x))


def _tc_body(x_ref, a_ref, b_ref, c_ref, w1_ref, b1_ref, w2_ref, b2_ref,
             w3_ref, b3_ref, wd_ref, bd_ref, irf_ref, d_ref):
    th = x_ref[...]
    c_s = _sigmoid(c_ref[...])
    theta_t = _VALUE_RANGE * (_sigmoid(th) - 0.5)
    b_t = _VALUE_RANGE * (_sigmoid(b_ref[...]) - 0.5)
    a_t = _A_RANGE * _sigmoid(a_ref[...])
    irf_ref[...] = c_s + (1.0 - c_s) / (
        1.0 + jnp.exp(-_D * a_t * (theta_t - b_t)))
    x = th.reshape(_B, 1)
    h1 = jnp.maximum(x * w1_ref[...] + b1_ref[...], 0.0)
    h2 = jnp.maximum(
        jnp.dot(h1, w2_ref[...], preferred_element_type=jnp.float32)
        + b2_ref[...], 0.0)
    h3 = lax.dot_general(
        w3_ref[...], h2, (((1,), (1,)), ((), ())),
        preferred_element_type=jnp.float32).reshape(_B)
    d_ref[...] = (h3 + b3_ref[0, 0]) * wd_ref[0, 0] + bd_ref[0, 0]


_tc_math = pl.pallas_call(
    _tc_body,
    out_shape=[
        jax.ShapeDtypeStruct((_B,), jnp.float32),
        jax.ShapeDtypeStruct((_B,), jnp.float32),
    ],
)


def kernel(user, item, theta_table, a_table, b_table, c_table,
           W1, b1, W2, b2, W3, b3, Wd, bd):
    af = a_table.reshape(-1)
    bf = b_table.reshape(-1)
    cf = c_table.reshape(-1)
    af, bf, cf, theta_table = lax.optimization_barrier(
        (af, bf, cf, theta_table))
    a_g, b_g, c_g = _sc_abc(item, af, bf, cf)
    theta_raw = _sc_theta(user, theta_table.reshape(-1))
    irf_out, d_out = _tc_math(
        theta_raw, a_g, b_g, c_g,
        W1.reshape(1, 64), b1.reshape(1, 64),
        W2.T, b2.reshape(1, 32),
        W3, b3.reshape(1, 1),
        Wd, bd.reshape(1, 1))
    return (irf_out, d_out.reshape(_B, 1))
